# Initial kernel scaffold; baseline (speedup 1.0000x reference)
import jax
import jax.numpy as jnp
from jax.experimental import pallas as pl


def kernel(vertices, faces, attributes):
    raise NotImplementedError("write your pallas kernel here")


# speedup vs baseline: 4.3109x; 4.3109x over previous
import jax
import jax.numpy as jnp
from jax.experimental import pallas as pl
from jax.experimental.pallas import tpu as pltpu

_H = 256
_W = 256
_FC = 128
_ROWS_PER_BLOCK = 8
_P = _ROWS_PER_BLOCK * _W


def _raster_body(fd_ref, attr_ref, out_ref):
    nchunks = fd_ref.shape[0]
    i = pl.program_id(0)

    pix = jax.lax.broadcasted_iota(jnp.int32, (_P, 1), 0) + i * _P
    row = pix // _W
    col = pix - row * _W
    pyf = 1.0 - 2.0 * (row.astype(jnp.float32) + 0.5) / _H
    pxf = 1.0 - 2.0 * (col.astype(jnp.float32) + 0.5) / _W
    lane = jax.lax.broadcasted_iota(jnp.int32, (_P, _FC), 1)
    inf = jnp.float32(jnp.inf)

    def body(c, carry):
        run_z, run_attr = carry
        fd = fd_ref[c]
        x0 = fd[0:1]
        y0 = fd[1:2]
        z0 = fd[2:3]
        x1 = fd[3:4]
        y1 = fd[4:5]
        z1 = fd[5:6]
        x2 = fd[6:7]
        y2 = fd[7:8]
        z2 = fd[8:9]
        den = fd[9:10]
        absarea = fd[10:11]

        w0 = ((x2 - x1) * (pyf - y1) - (y2 - y1) * (pxf - x1)) / den
        w1 = ((x0 - x2) * (pyf - y2) - (y0 - y2) * (pxf - x2)) / den
        w2 = ((x1 - x0) * (pyf - y0) - (y1 - y0) * (pxf - x0)) / den
        inside = (w0 >= 0.0) & (w1 >= 0.0) & (w2 >= 0.0) & (absarea > 1e-8)
        zb = jnp.where(inside, w0 * z0 + w1 * z1 + w2 * z2, inf)

        minz = jnp.min(zb, axis=1, keepdims=True)
        cand = jnp.where(zb == minz, lane, _FC)
        lanewin = jnp.min(cand, axis=1, keepdims=True)
        onehot = (lane == lanewin).astype(jnp.float32)

        wcat = jnp.concatenate([onehot * w0, onehot * w1, onehot * w2],
                               axis=1)
        attr_c = attr_ref[c]
        contrib = jnp.dot(wcat, attr_c,
                          preferred_element_type=jnp.float32)

        better = minz < run_z
        run_z = jnp.where(better, minz, run_z)
        run_attr = jnp.where(better, contrib, run_attr)
        return run_z, run_attr

    run_z0 = jnp.full((_P, 1), inf, dtype=jnp.float32)
    run_attr0 = jnp.zeros((_P, 8), dtype=jnp.float32)
    run_z, run_attr = jax.lax.fori_loop(0, nchunks, body, (run_z0, run_attr0))

    hit = run_z < inf
    out_ref[...] = jnp.concatenate(
        [jnp.where(hit, run_attr, 0.0), hit.astype(jnp.float32),
         jnp.zeros((_P, 7), dtype=jnp.float32)], axis=1)


def kernel(vertices, faces, attributes):
    verts = vertices[0].astype(jnp.float32)
    f = faces[0]
    F = f.shape[0]
    D = attributes.shape[-1]

    fv = verts[f]
    x0, y0, z0 = fv[:, 0, 0], fv[:, 0, 1], fv[:, 0, 2]
    x1, y1, z1 = fv[:, 1, 0], fv[:, 1, 1], fv[:, 1, 2]
    x2, y2, z2 = fv[:, 2, 0], fv[:, 2, 1], fv[:, 2, 2]
    area = (x1 - x0) * (y2 - y0) - (y1 - y0) * (x2 - x0)
    den = jnp.where(jnp.abs(area) > 1e-8, area, 1.0)
    absarea = jnp.abs(area)

    Fp = ((F + _FC - 1) // _FC) * _FC
    nchunks = Fp // _FC
    pad = Fp - F

    def padf(a, val):
        return jnp.pad(a, (0, pad), constant_values=val)

    fd = jnp.stack([
        padf(x0, 0.0), padf(y0, 0.0), padf(z0, 0.0),
        padf(x1, 0.0), padf(y1, 0.0), padf(z1, 0.0),
        padf(x2, 0.0), padf(y2, 0.0), padf(z2, 0.0),
        padf(den, 1.0), padf(absarea, 0.0),
        jnp.zeros((Fp,), jnp.float32), jnp.zeros((Fp,), jnp.float32),
        jnp.zeros((Fp,), jnp.float32), jnp.zeros((Fp,), jnp.float32),
        jnp.zeros((Fp,), jnp.float32),
    ], axis=0)
    fd = fd.reshape(16, nchunks, _FC).transpose(1, 0, 2)

    attr = attributes[0].astype(jnp.float32)
    attr = jnp.pad(attr, ((0, pad), (0, 0), (0, 0)))
    attrM = attr.reshape(nchunks, _FC, 3, D).transpose(0, 2, 1, 3)
    attrM = attrM.reshape(nchunks, 3 * _FC, D)

    nblocks = (_H * _W) // _P
    out = pl.pallas_call(
        _raster_body,
        grid=(nblocks,),
        in_specs=[
            pl.BlockSpec((nchunks, 16, _FC), lambda i: (0, 0, 0)),
            pl.BlockSpec((nchunks, 3 * _FC, D), lambda i: (0, 0, 0)),
        ],
        out_specs=pl.BlockSpec((_P, 16), lambda i: (i, 0)),
        out_shape=jax.ShapeDtypeStruct((_H * _W, 16), jnp.float32),
    )(fd, attrM)

    img = out[:, 0:9].reshape(_H, _W, 9).transpose(2, 0, 1)
    return img[None]


# speedup vs baseline: 5.1251x; 1.1889x over previous
import functools

import jax
import jax.numpy as jnp
from jax import lax
from jax.experimental import pallas as pl
from jax.experimental.pallas import tpu as pltpu
from jax.experimental.pallas import tpu_sc as plsc

_H = 256
_W = 256
_FC = 128
_ROWS_PER_BLOCK = 8
_P = _ROWS_PER_BLOCK * _W
_D = 8

_SC_CHUNK = 128


def _pix_coords(i):
    pix = lax.broadcasted_iota(jnp.int32, (_P, 1), 0) + i * _P
    row = pix >> 8
    col = pix & (_W - 1)
    pyf = 1.0 - 2.0 * (row.astype(jnp.float32) + 0.5) / _H
    pxf = 1.0 - 2.0 * (col.astype(jnp.float32) + 0.5) / _W
    return pxf, pyf


def _raster_body(fd_ref, idx_ref, vis_ref):
    nchunks = fd_ref.shape[0]
    pxf, pyf = _pix_coords(pl.program_id(0))
    lane = lax.broadcasted_iota(jnp.int32, (_P, _FC), 1)
    inf = jnp.float32(jnp.inf)

    def body(c, carry):
        run_z, run_i = carry
        fd = fd_ref[c]
        na0, nb0, nc0 = fd[0:1], fd[1:2], fd[2:3]
        na1, nb1, nc1 = fd[3:4], fd[4:5], fd[5:6]
        na2, nb2, nc2 = fd[6:7], fd[7:8], fd[8:9]
        za, zbx, zc = fd[9:10], fd[10:11], fd[11:12]

        n0 = na0 * pyf + nb0 * pxf + nc0
        n1 = na1 * pyf + nb1 * pxf + nc1
        n2 = na2 * pyf + nb2 * pxf + nc2
        inside = jnp.minimum(jnp.minimum(n0, n1), n2) >= 0.0
        zb = za * pyf + zbx * pxf + zc
        zf = jnp.where(inside, zb, inf)

        minz = jnp.min(zf, axis=1, keepdims=True)
        cand = jnp.where(zf == minz, lane, _FC)
        lanewin = jnp.min(cand, axis=1, keepdims=True)

        better = minz < run_z
        run_z = jnp.where(better, minz, run_z)
        run_i = jnp.where(better, lanewin + c * _FC, run_i)
        return run_z, run_i

    run_z0 = jnp.full((_P, 1), inf, dtype=jnp.float32)
    run_i0 = jnp.full((_P, 1), -1, dtype=jnp.int32)
    run_z, run_i = lax.fori_loop(0, nchunks, body, (run_z0, run_i0))

    idx_ref[...] = jnp.maximum(run_i, 0)
    vis_ref[...] = (run_z < inf).astype(jnp.float32)


def _gather_body(coef_hbm, idx_hbm, g_hbm, idx_v, rows_v, sem):
    wid = lax.axis_index("s") * 2 + lax.axis_index("c")
    npix = (_H * _W) // 32
    nchunks = npix // _SC_CHUNK

    def chunk(ci, carry):
        base = wid * npix + ci * _SC_CHUNK
        pltpu.sync_copy(idx_hbm.at[pl.ds(base, _SC_CHUNK)], idx_v)
        pltpu.async_copy(coef_hbm.at[idx_v], rows_v, sem).wait()
        pltpu.sync_copy(rows_v, g_hbm.at[pl.ds(base, _SC_CHUNK)])
        return carry

    lax.fori_loop(0, nchunks, chunk, 0)


def _interp_body(g_ref, vis_ref, out_ref):
    pxf, pyf = _pix_coords(pl.program_id(0))
    g = g_ref[...]
    gp = g[:, 0:_D]
    gq = g[:, _D:2 * _D]
    gr = g[:, 2 * _D:3 * _D]
    vis = vis_ref[...]
    out8 = (gp * pyf + gq * pxf + gr) * vis
    out_ref[...] = jnp.concatenate(
        [out8, vis, jnp.zeros((_P, 16 - _D - 1), jnp.float32)], axis=1)


def kernel(vertices, faces, attributes):
    verts = vertices[0].astype(jnp.float32)
    f = faces[0]
    F = f.shape[0]

    fv = verts[f]
    x0, y0, z0 = fv[:, 0, 0], fv[:, 0, 1], fv[:, 0, 2]
    x1, y1, z1 = fv[:, 1, 0], fv[:, 1, 1], fv[:, 1, 2]
    x2, y2, z2 = fv[:, 2, 0], fv[:, 2, 1], fv[:, 2, 2]
    area = (x1 - x0) * (y2 - y0) - (y1 - y0) * (x2 - x0)
    valid = jnp.abs(area) > 1e-8
    den = jnp.where(valid, area, 1.0)
    s = jnp.sign(den)

    e0x, e0y = x2 - x1, y2 - y1
    e1x, e1y = x0 - x2, y0 - y2
    e2x, e2y = x1 - x0, y1 - y0
    c0 = e0y * x1 - e0x * y1
    c1 = e1y * x2 - e1x * y2
    c2 = e2y * x0 - e2x * y0

    na0, nb0 = s * e0x, -s * e0y
    na1, nb1 = s * e1x, -s * e1y
    na2, nb2 = s * e2x, -s * e2y
    nc0 = jnp.where(valid, s * c0, -1.0)
    na0 = jnp.where(valid, na0, 0.0)
    nb0 = jnp.where(valid, nb0, 0.0)
    nc1, nc2 = s * c1, s * c2
    za = (e0x * z0 + e1x * z1 + e2x * z2) / den
    zbx = -(e0y * z0 + e1y * z1 + e2y * z2) / den
    zc = (c0 * z0 + c1 * z1 + c2 * z2) / den

    Fp = ((F + _FC - 1) // _FC) * _FC
    nchunks = Fp // _FC
    pad = Fp - F

    def padf(a):
        return jnp.pad(a, (0, pad))

    zero = jnp.zeros((Fp,), jnp.float32)
    fd = jnp.stack([
        padf(na0), padf(nb0), jnp.pad(nc0, (0, pad), constant_values=-1.0),
        padf(na1), padf(nb1), padf(nc1),
        padf(na2), padf(nb2), padf(nc2),
        padf(za), padf(zbx), padf(zc),
        zero, zero, zero, zero,
    ], axis=0)
    fd = fd.reshape(16, nchunks, _FC).transpose(1, 0, 2)

    att = attributes[0].astype(jnp.float32)
    ex = jnp.stack([e0x, e1x, e2x], 1)
    ey = jnp.stack([e0y, e1y, e2y], 1)
    cc = jnp.stack([c0, c1, c2], 1)
    Pm = jnp.einsum('fk,fkd->fd', ex, att) / den[:, None]
    Qm = -jnp.einsum('fk,fkd->fd', ey, att) / den[:, None]
    Rm = jnp.einsum('fk,fkd->fd', cc, att) / den[:, None]
    coef = jnp.concatenate([Pm, Qm, Rm], axis=1)

    nblocks = (_H * _W) // _P
    idx, vis = pl.pallas_call(
        _raster_body,
        grid=(nblocks,),
        in_specs=[
            pl.BlockSpec((nchunks, 16, _FC), lambda i: (0, 0, 0)),
        ],
        out_specs=[
            pl.BlockSpec((_P, 1), lambda i: (i, 0)),
            pl.BlockSpec((_P, 1), lambda i: (i, 0)),
        ],
        out_shape=[
            jax.ShapeDtypeStruct((_H * _W, 1), jnp.int32),
            jax.ShapeDtypeStruct((_H * _W, 1), jnp.float32),
        ],
    )(fd)

    idx1 = idx.reshape(_H * _W)

    mesh = plsc.VectorSubcoreMesh(core_axis_name="c", subcore_axis_name="s")
    gathered = functools.partial(
        pl.kernel, mesh=mesh,
        out_type=jax.ShapeDtypeStruct((_H * _W, 3 * _D), jnp.float32),
        compiler_params=pltpu.CompilerParams(use_tc_tiling_on_sc=False),
        scratch_types=[
            pltpu.VMEM((_SC_CHUNK,), jnp.int32),
            pltpu.VMEM((_SC_CHUNK, 3 * _D), jnp.float32),
            pltpu.SemaphoreType.DMA,
        ],
    )(_gather_body)(coef, idx1)

    out = pl.pallas_call(
        _interp_body,
        grid=(nblocks,),
        in_specs=[
            pl.BlockSpec((_P, 3 * _D), lambda i: (i, 0)),
            pl.BlockSpec((_P, 1), lambda i: (i, 0)),
        ],
        out_specs=pl.BlockSpec((_P, 16), lambda i: (i, 0)),
        out_shape=jax.ShapeDtypeStruct((_H * _W, 16), jnp.float32),
    )(gathered, vis)

    img = out[:, 0:_D + 1].reshape(_H, _W, _D + 1).transpose(2, 0, 1)
    return img[None]
